# Initial kernel scaffold; baseline (speedup 1.0000x reference)
import jax
import jax.numpy as jnp
from jax.experimental import pallas as pl


def kernel(out, downsample_ratio, gt_kernels, training_masks, gt_instances, gt_kernel_instances, training_mask_distances, gt_distances):
    raise NotImplementedError("write your pallas kernel here")


# speedup vs baseline: 11.6167x; 11.6167x over previous
import functools

import jax
import jax.numpy as jnp
from jax import lax
from jax.experimental import pallas as pl
from jax.experimental.pallas import tpu as pltpu
from jax.experimental.pallas import tpu_sc as plsc

import numpy as np

B, H, W = 4, 640, 640
NPIX = H * W
INT_MIN = np.int32(-2 ** 31)
MAGIC = np.int32(0x7FFFFFFF)


def _ohem_dice_body(s_ref, gt_ref, tm_ref, out_ref):
    s = s_ref[0]
    gt = gt_ref[0]
    tm = tm_ref[0]

    tm_pos = tm > 0.5
    gt_pos = gt > 0
    pos = jnp.sum((gt_pos & tm_pos).astype(jnp.int32))
    negmask = (~gt_pos) & tm_pos
    neg_total = jnp.sum(negmask.astype(jnp.int32))
    neg_num = jnp.minimum(pos * 3, neg_total)
    k = jnp.maximum(neg_num, 1)

    bits = lax.bitcast_convert_type(s, jnp.int32)
    key = bits ^ ((bits >> 31) & MAGIC)
    key = jnp.where(negmask, key, INT_MIN)

    c0 = jnp.sum((key >= 0).astype(jnp.int32))
    v0 = jnp.where(c0 >= k, jnp.int32(0), INT_MIN)

    def bit_body(t, v):
        vt = v | (jnp.int32(1) << (30 - t))
        c = jnp.sum((key >= vt).astype(jnp.int32))
        return jnp.where(c >= k, vt, v)

    v = lax.fori_loop(0, 31, bit_body, v0)
    tb = jnp.where(v >= 0, v, v ^ MAGIC)
    thr = lax.bitcast_convert_type(tb, jnp.float32)

    fallback = (pos == 0) | (neg_num == 0)
    selm = ((s >= thr) | gt_pos) & tm_pos
    sel = jnp.where(fallback, tm, selm.astype(jnp.float32))

    sig = jax.nn.sigmoid(s)
    inp = sig * sel
    tgt = gt.astype(jnp.float32) * sel
    a = jnp.sum(inp * tgt)
    b2 = jnp.sum(inp * inp) + 0.001
    c2 = jnp.sum(tgt * tgt) + 0.001
    loss = 1.0 - 2.0 * a / (b2 + c2)
    out_ref[0, 0, :] = jnp.full((128,), loss, jnp.float32)


def _ohem_dice_call(score, gt_kernels, training_masks, interpret=False):
    return pl.pallas_call(
        _ohem_dice_body,
        grid=(B,),
        in_specs=[
            pl.BlockSpec((1, H, W), lambda i: (i, 0, 0)),
            pl.BlockSpec((1, H, W), lambda i: (i, 0, 0)),
            pl.BlockSpec((1, H, W), lambda i: (i, 0, 0)),
        ],
        out_specs=pl.BlockSpec((1, 1, 128), lambda i: (i, 0, 0)),
        out_shape=jax.ShapeDtypeStruct((B, 1, 128), jnp.float32),
        interpret=interpret,
    )(score, gt_kernels, training_masks)


_TILES = 32
_ROWS_PER_TILE = 80
_CHUNK_ROWS = 16
_CHUNKS = _ROWS_PER_TILE // _CHUNK_ROWS
_CPX = _CHUNK_ROWS * W
_GROWS = _CPX // 128


def _loc_body(d0, d1, g0, g1, tmd, gi, tab, out_hbm,
              vd0, vd1, vg0, vg1, vtmd, vgi, vidx, vgath, vout, sem1, sem2):
    nc = 2
    wid = lax.axis_index("s") * nc + lax.axis_index("c")
    img = wid >> 3
    row0 = (wid - (img << 3)) * _ROWS_PER_TILE
    tile_base = wid * (_ROWS_PER_TILE * W)
    img_base = img * NPIX

    lanes = lax.iota(jnp.int32, 16)

    def chunk_body(c, accs):
        acc_l, acc_m = accs
        base = tile_base + c * _CPX
        copies = [
            pltpu.async_copy(d0.at[pl.ds(base, _CPX)], vd0, sem1),
            pltpu.async_copy(d1.at[pl.ds(base, _CPX)], vd1, sem1),
            pltpu.async_copy(g0.at[pl.ds(base, _CPX)], vg0, sem1),
            pltpu.async_copy(g1.at[pl.ds(base, _CPX)], vg1, sem1),
            pltpu.async_copy(tmd.at[pl.ds(base, _CPX)], vtmd, sem1),
            pltpu.async_copy(gi.at[pl.ds(base, _CPX)], vgi, sem1),
        ]
        for h in copies:
            h.wait()

        row_chunk0 = row0 + c * _CHUNK_ROWS

        def idx_row(rr, _):
            rowf = jnp.full((16,), row_chunk0 + rr, jnp.int32).astype(jnp.float32)

            def idx_col(cb, _):
                v = rr * (W // 16) + cb
                off = v * 16
                colf = (cb * 16 + lanes).astype(jnp.float32)
                dx = vd0[pl.ds(off, 16)]
                dy = vd1[pl.ds(off, 16)]
                ox = jnp.clip((colf + 10.0 * dx).astype(jnp.int32), 0, W - 1)
                oy = jnp.clip((rowf + 10.0 * dy).astype(jnp.int32), 0, H - 1)
                vidx[v >> 3, pl.ds((v & 7) * 16, 16)] = img_base + oy * W + ox
                return 0

            lax.fori_loop(0, W // 16, idx_col, 0)
            return 0

        lax.fori_loop(0, _CHUNK_ROWS, idx_row, 0)

        def fire(j, _):
            pltpu.async_copy(tab.at[vidx.at[j]], vgath.at[pl.ds(j * 128, 128)], sem2)
            return 0

        lax.fori_loop(0, _GROWS, fire, 0)
        pltpu.make_async_copy(tab.at[pl.ds(0, _CPX)], vgath, sem2).wait()

        def px_body(v, accs):
            acc_l, acc_m = accs
            off = v * 16
            neq = vgath[pl.ds(off, 16)] != vgi[pl.ds(off, 16)]
            selt = jnp.where(neq, 1.0, 0.0).astype(jnp.float32) * vtmd[pl.ds(off, 16)]
            diff0 = jnp.abs(vd0[pl.ds(off, 16)] - vg0[pl.ds(off, 16)]) * selt
            diff1 = jnp.abs(vd1[pl.ds(off, 16)] - vg1[pl.ds(off, 16)]) * selt
            l0 = jnp.where(diff0 < 0.1, 5.0 * diff0 * diff0, diff0 - 0.05)
            l1 = jnp.where(diff1 < 0.1, 5.0 * diff1 * diff1, diff1 - 0.05)
            return acc_l + l0 + l1, acc_m + selt

        acc_l, acc_m = lax.fori_loop(0, _CPX // 16, px_body, (acc_l, acc_m))
        return acc_l, acc_m

    zero = jnp.zeros((16,), jnp.float32)
    acc_l, acc_m = lax.fori_loop(0, _CHUNKS, chunk_body, (zero, zero))
    vout[0, :] = acc_l
    vout[1, :] = acc_m
    pltpu.sync_copy(vout, out_hbm.at[wid])


def _loc_call(d0, d1, g0, g1, tmd, gi, tab):
    mesh = plsc.VectorSubcoreMesh(core_axis_name="c", subcore_axis_name="s")
    f = functools.partial(
        pl.kernel,
        out_type=jax.ShapeDtypeStruct((_TILES, 2, 16), jnp.float32),
        mesh=mesh,
        scratch_types=[
            pltpu.VMEM((_CPX,), jnp.float32),
            pltpu.VMEM((_CPX,), jnp.float32),
            pltpu.VMEM((_CPX,), jnp.float32),
            pltpu.VMEM((_CPX,), jnp.float32),
            pltpu.VMEM((_CPX,), jnp.float32),
            pltpu.VMEM((_CPX,), jnp.int32),
            pltpu.VMEM((_GROWS, 128), jnp.int32),
            pltpu.VMEM((_CPX,), jnp.int32),
            pltpu.VMEM((2, 16), jnp.float32),
            pltpu.SemaphoreType.DMA,
            pltpu.SemaphoreType.DMA,
        ],
    )(_loc_body)
    return f(d0, d1, g0, g1, tmd, gi, tab)


def kernel(out, downsample_ratio, gt_kernels, training_masks, gt_instances,
           gt_kernel_instances, training_mask_distances, gt_distances):
    out = out * jnp.asarray(downsample_ratio, out.dtype)
    score = out[:, 0]
    d0 = out[:, 1].reshape(-1)
    d1 = out[:, 2].reshape(-1)
    g0 = gt_distances[:, 0].reshape(-1)
    g1 = gt_distances[:, 1].reshape(-1)
    tmd = training_mask_distances.reshape(-1)
    gi = gt_instances.reshape(-1)
    tab = gt_kernel_instances.reshape(-1)

    loss_kernel = _ohem_dice_call(score, gt_kernels, training_masks)[:, 0, 0]

    partials = _loc_call(d0, d1, g0, g1, tmd, gi, tab)
    p = partials.reshape(B, 8, 2, 16)
    s_loss = jnp.sum(p[:, :, 0, :], axis=(1, 2))
    s_mask = jnp.sum(p[:, :, 1, :], axis=(1, 2))
    loss_loc = 0.05 * s_loss / (s_mask + 1e-6)
    return (loss_kernel, loss_loc)
